# Initial kernel scaffold; baseline (speedup 1.0000x reference)
import jax
import jax.numpy as jnp
from jax.experimental import pallas as pl


def kernel(logits):
    raise NotImplementedError("write your pallas kernel here")


# speedup vs baseline: 2.1561x; 2.1561x over previous
import functools

import jax
import jax.numpy as jnp
from jax import lax
from jax.experimental import pallas as pl
from jax.experimental.pallas import tpu as pltpu
from jax.experimental.pallas import tpu_sc as plsc

B = 64
N = 1_000_000
K = 100
KPAD = 112
ACTW = 16
W = 20_000
NWIN_PAIRS = (N // W) // 2
G = 25
NGROUPS = W // (16 * G)
CAP = 4096
FCAP = 256
ROWS_PER_WORKER = 2
NEG_INF = float("-inf")


def _lanes():
    return lax.iota(jnp.int32, 16)


def _splat_f32(x):
    return jnp.full((16,), x, dtype=jnp.float32)


def _splat_i32(x):
    return jnp.full((16,), x, dtype=jnp.int32)


def _mono_u32(x):
    b = plsc.bitcast(x, jnp.uint32)
    neg = b >= jnp.uint32(0x80000000)
    flip = jnp.where(neg, jnp.uint32(0xFFFFFFFF), jnp.uint32(0x80000000))
    return b ^ flip


def _sc_topk_sample(logits_flat, gumbel_flat):
    mesh = plsc.VectorSubcoreMesh(core_axis_name="c", subcore_axis_name="s")

    @functools.partial(
        pl.kernel,
        out_type=(
            jax.ShapeDtypeStruct((B * KPAD,), jnp.float32),
            jax.ShapeDtypeStruct((B * ACTW,), jnp.int32),
        ),
        mesh=mesh,
        compiler_params=pltpu.CompilerParams(needs_layout_passes=False),
        scratch_types=[
            pltpu.VMEM((W,), jnp.float32),
            pltpu.VMEM((W,), jnp.float32),
            pltpu.VMEM((CAP + 16,), jnp.float32),
            pltpu.VMEM((CAP + 16,), jnp.int32),
            pltpu.VMEM((FCAP,), jnp.float32),
            pltpu.VMEM((FCAP,), jnp.int32),
            pltpu.VMEM((KPAD,), jnp.float32),
            pltpu.VMEM((KPAD,), jnp.int32),
            pltpu.VMEM((KPAD,), jnp.float32),
            pltpu.VMEM((ACTW,), jnp.int32),
            pltpu.SemaphoreType.DMA,
            pltpu.SemaphoreType.DMA,
        ],
    )
    def k(logits_hbm, gumbel_hbm, vals_hbm, act_hbm,
          win0, win1, cv, ci, fv, fi, ov, oi, gb, actb, sem0, sem1):
        lanes = _lanes()
        wid = lax.axis_index("s") * 2 + lax.axis_index("c")

        def count_above(mid_u, cnt_v, nj):
            def cbody(j, acc):
                x = cv[pl.ds(j * 16, 16)]
                u = _mono_u32(x)
                valid = (j * 16 + lanes) < cnt_v
                m = jnp.logical_and(u > mid_u, valid)
                return acc + plsc.all_reduce_population_count(m)
            return lax.fori_loop(0, nj, cbody, _splat_i32(0))

        def find_threshold(cnt_v, hi_target):
            nj = (jnp.max(cnt_v) + 15) >> 4
            c0 = jnp.max(cnt_v)

            def cond(s):
                lo, hi, c = s
                return jnp.logical_and(hi - lo > jnp.uint32(1),
                                       jnp.logical_or(c < K, c > hi_target))

            def body(s):
                lo, hi, c = s
                mid = lo + ((hi - lo) >> jnp.uint32(1))
                cm = jnp.max(count_above(jnp.full((16,), mid, jnp.uint32),
                                         cnt_v, nj))
                ok = cm >= K
                lo2 = jnp.where(ok, mid, lo)
                hi2 = jnp.where(ok, hi, mid)
                c2 = jnp.where(ok, cm, c)
                return (lo2, hi2, c2)

            lo, _, c = lax.while_loop(
                cond, body,
                (jnp.uint32(0), jnp.uint32(0xFFFFFFFF), c0))
            return jnp.full((16,), lo, jnp.uint32), _splat_i32(1) * c

        def compress_into(dst_v, dst_i, dcap, t_u, cnt_v):
            nj = (jnp.max(cnt_v) + 15) >> 4

            def cpbody(j, newcnt):
                x = cv[pl.ds(j * 16, 16)]
                col = ci[pl.ds(j * 16, 16)]
                valid = (j * 16 + lanes) < cnt_v
                m = jnp.logical_and(_mono_u32(x) > t_u, valid)
                ones = jnp.where(m, 1, 0).astype(jnp.int32)
                incl = plsc.cumsum(ones)
                pos = jnp.minimum(newcnt + incl - 1, dcap - 1)
                plsc.store_scatter(dst_v, [pos], x, mask=m)
                plsc.store_scatter(dst_i, [pos], col, mask=m)
                return jnp.minimum(newcnt + plsc.all_reduce_population_count(m),
                                   _splat_i32(dcap))
            return lax.fori_loop(0, nj, cpbody, _splat_i32(0))

        def compact(cnt_v, tvec):
            t_u, _ = find_threshold(cnt_v, 1024)
            newcnt = compress_into(cv, ci, CAP, t_u, cnt_v)
            bv = t_u ^ jnp.where(t_u >= jnp.uint32(0x80000000),
                                 jnp.uint32(0x80000000),
                                 jnp.uint32(0xFFFFFFFF))
            return newcnt, plsc.bitcast(bv, jnp.float32)

        def scan_window(win, base_col, cnt_v, tvec):
            def gbody(g, carry):
                cnt_v, tvec = carry
                base = g * (G * 16)
                anym = win[pl.ds(base, 16)] > tvec
                for i in range(1, G):
                    anym = jnp.logical_or(
                        anym, win[pl.ds(base + i * 16, 16)] > tvec)

                def slow(cnt_v):
                    for i in range(G):
                        x = win[pl.ds(base + i * 16, 16)]
                        m = x > tvec
                        colv = base_col + base + i * 16 + lanes
                        ones = jnp.where(m, 1, 0).astype(jnp.int32)
                        incl = plsc.cumsum(ones)
                        pos = jnp.minimum(cnt_v + incl - 1,
                                          _splat_i32(CAP + 15))
                        plsc.store_scatter(cv, [pos], x, mask=m)
                        plsc.store_scatter(ci, [pos], colv, mask=m)
                        cnt_v = jnp.minimum(
                            cnt_v + plsc.all_reduce_population_count(m),
                            _splat_i32(CAP))
                    return cnt_v

                cnt_v = lax.cond(jnp.any(anym), slow, lambda c: c, cnt_v)
                cnt_v, tvec = lax.cond(
                    jnp.any(cnt_v >= CAP - 16 * G),
                    compact, lambda c, t: (c, t), cnt_v, tvec)
                return cnt_v, tvec
            return lax.fori_loop(0, NGROUPS, gbody, (cnt_v, tvec))

        def extract_topk():
            def ebody(j, _):
                mvec = fv[pl.ds(0, 16)]
                for i in range(1, FCAP // 16):
                    mvec = jnp.maximum(mvec, fv[pl.ds(i * 16, 16)])
                msp = _splat_f32(1.0) * jnp.max(mvec)
                pos_v = _splat_i32(FCAP)
                for i in range(FCAP // 16):
                    eq = fv[pl.ds(i * 16, 16)] == msp
                    pos_v = jnp.minimum(
                        pos_v, jnp.where(eq, i * 16 + lanes, FCAP))
                pos = _splat_i32(1) * jnp.min(pos_v)
                pos = jnp.minimum(pos, _splat_i32(FCAP - 1))
                idv = plsc.load_gather(fi, [pos])
                jsp = _splat_i32(1) * j
                lane0 = lanes == 0
                plsc.store_scatter(ov, [jsp], msp, mask=lane0)
                plsc.store_scatter(oi, [jsp], idv, mask=lane0)
                plsc.store_scatter(fv, [pos], _splat_f32(NEG_INF), mask=lane0)
                return 0
            lax.fori_loop(0, K, ebody, 0)

        def sample_row():
            zbest = _splat_f32(NEG_INF)
            zs = []
            for i in range(KPAD // 16):
                z = ov[pl.ds(i * 16, 16)] + gb[pl.ds(i * 16, 16)]
                zs.append(z)
                zbest = jnp.maximum(zbest, z)
            msp = _splat_f32(1.0) * jnp.max(zbest)
            pos_v = _splat_i32(KPAD)
            for i in range(KPAD // 16):
                eq = zs[i] == msp
                pos_v = jnp.minimum(
                    pos_v, jnp.where(eq, i * 16 + lanes, KPAD))
            pos = _splat_i32(1) * jnp.min(pos_v)
            pos = jnp.minimum(pos, _splat_i32(KPAD - 1))
            return plsc.load_gather(oi, [pos])

        def row_body(r, _):
            row = wid * ROWS_PER_WORKER + r
            rbase = row * N
            pltpu.make_async_copy(
                logits_hbm.at[pl.ds(rbase, W)], win0, sem0).start()

            def wbody(i, carry):
                cnt_v, tvec = carry
                w0 = 2 * i
                pltpu.make_async_copy(
                    logits_hbm.at[pl.ds(rbase + (w0 + 1) * W, W)],
                    win1, sem1).start()
                pltpu.make_async_copy(
                    logits_hbm.at[pl.ds(rbase + w0 * W, W)],
                    win0, sem0).wait()
                cnt_v, tvec = scan_window(win0, w0 * W, cnt_v, tvec)

                @pl.when(i < NWIN_PAIRS - 1)
                def _():
                    pltpu.make_async_copy(
                        logits_hbm.at[pl.ds(rbase + (w0 + 2) * W, W)],
                        win0, sem0).start()

                pltpu.make_async_copy(
                    logits_hbm.at[pl.ds(rbase + (w0 + 1) * W, W)],
                    win1, sem1).wait()
                cnt_v, tvec = scan_window(win1, (w0 + 1) * W, cnt_v, tvec)
                return cnt_v, tvec

            cnt_v, _ = lax.fori_loop(
                0, NWIN_PAIRS, wbody,
                (_splat_i32(0), _splat_f32(NEG_INF)))

            t_u, _ = find_threshold(cnt_v, FCAP - 16)
            for i in range(FCAP // 16):
                fv[pl.ds(i * 16, 16)] = _splat_f32(NEG_INF)
                fi[pl.ds(i * 16, 16)] = _splat_i32(0)
            compress_into(fv, fi, FCAP, t_u, cnt_v)
            for i in range(KPAD // 16):
                ov[pl.ds(i * 16, 16)] = _splat_f32(NEG_INF)
                oi[pl.ds(i * 16, 16)] = _splat_i32(0)
            extract_topk()
            pltpu.sync_copy(gumbel_hbm.at[pl.ds(row * KPAD, KPAD)], gb)
            actb[...] = sample_row()
            pltpu.sync_copy(ov, vals_hbm.at[pl.ds(row * KPAD, KPAD)])
            pltpu.sync_copy(actb, act_hbm.at[pl.ds(row * ACTW, ACTW)])
            return 0

        lax.fori_loop(0, ROWS_PER_WORKER, row_body, 0)

    return k(logits_flat, gumbel_flat)


def kernel(logits):
    gkey = jax.random.key(42)
    u = jax.random.uniform(gkey, (B, K), minval=1e-20, maxval=1.0)
    gumbel = -jnp.log(-jnp.log(u))
    gpad = jnp.full((B, KPAD), NEG_INF, dtype=jnp.float32)
    gpad = gpad.at[:, :K].set(gumbel)

    vals_flat, act_flat = _sc_topk_sample(
        logits.reshape(-1), gpad.reshape(-1))
    vals = vals_flat.reshape(B, KPAD)[:, :K]
    act = act_flat.reshape(B, ACTW)[:, 0]
    return act, vals


# speedup vs baseline: 10.6194x; 4.9252x over previous
import functools

import jax
import jax.numpy as jnp
from jax import lax
from jax.experimental import pallas as pl
from jax.experimental.pallas import tpu as pltpu
from jax.experimental.pallas import tpu_sc as plsc

B = 64
N = 1_000_000
K = 100
KPAD = 112
ACTW = 16
CT = 6
WC = 128 * CT
NFULL = (N // 128) // CT
NWIN_PAIRS = NFULL // 2
TAILC = N - NFULL * WC
CAP = 4096
FCAP = 128
NEG_INF = float("-inf")


def _lanes():
    return lax.iota(jnp.int32, 16)


def _splat_f32(x):
    return jnp.full((16,), x, dtype=jnp.float32)


def _splat_i32(x):
    return jnp.full((16,), x, dtype=jnp.int32)


def _mono_u32(x):
    b = plsc.bitcast(x, jnp.uint32)
    neg = b >= jnp.uint32(0x80000000)
    flip = jnp.where(neg, jnp.uint32(0xFFFFFFFF), jnp.uint32(0x80000000))
    return b ^ flip


def _sc_topk_sample(logits2d, tail_flat, gumbel_flat):
    mesh = plsc.VectorSubcoreMesh(core_axis_name="c", subcore_axis_name="s")

    @functools.partial(
        pl.kernel,
        out_type=(
            jax.ShapeDtypeStruct((B * KPAD,), jnp.float32),
            jax.ShapeDtypeStruct((B * ACTW,), jnp.int32),
        ),
        mesh=mesh,
        compiler_params=pltpu.CompilerParams(
            needs_layout_passes=False, use_tc_tiling_on_sc=True),
        scratch_types=[
            pltpu.VMEM((8, WC), jnp.float32),
            pltpu.VMEM((8, WC), jnp.float32),
            pltpu.VMEM((TAILC,), jnp.float32),
            pltpu.VMEM((CAP + 16,), jnp.float32),
            pltpu.VMEM((CAP + 16,), jnp.int32),
            pltpu.VMEM((CAP + 16,), jnp.float32),
            pltpu.VMEM((CAP + 16,), jnp.int32),
            pltpu.VMEM((FCAP,), jnp.float32),
            pltpu.VMEM((FCAP,), jnp.int32),
            pltpu.VMEM((KPAD,), jnp.float32),
            pltpu.VMEM((KPAD,), jnp.int32),
            pltpu.VMEM((KPAD,), jnp.float32),
            pltpu.VMEM((ACTW,), jnp.int32),
            pltpu.SemaphoreType.DMA,
            pltpu.SemaphoreType.DMA,
        ],
    )
    def k(logits_hbm, tail_hbm, gumbel_hbm, vals_hbm, act_hbm,
          win0, win1, wtail, cv0, ci0, cv1, ci1, fv, fi, ov, oi, gb, actb,
          sem0, sem1):
        lanes = _lanes()
        wid = lax.axis_index("s") * 2 + lax.axis_index("c")
        row0 = wid * 2
        g8 = pl.multiple_of((wid >> 2) * 8, 8)
        sub0 = (row0 % 8)
        cvs = (cv0, cv1)
        cis = (ci0, ci1)

        def count_above(src_v, mid_u, cnt_v, nj):
            def cbody(j, acc):
                x = src_v[pl.ds(j * 16, 16)]
                u = _mono_u32(x)
                valid = (j * 16 + lanes) < cnt_v
                m = jnp.logical_and(u > mid_u, valid)
                return acc + plsc.all_reduce_population_count(m)
            return lax.fori_loop(0, nj, cbody, _splat_i32(0))

        def find_threshold(src_v, cnt_v, hi_target):
            nj = (jnp.max(cnt_v) + 15) >> 4
            c0 = jnp.max(cnt_v)

            def cond(s):
                lo, hi, c = s
                return jnp.logical_and(hi - lo > jnp.uint32(1),
                                       jnp.logical_or(c < K, c > hi_target))

            def body(s):
                lo, hi, c = s
                mid = lo + ((hi - lo) >> jnp.uint32(1))
                cm = jnp.max(count_above(src_v,
                                         jnp.full((16,), mid, jnp.uint32),
                                         cnt_v, nj))
                ok = cm >= K
                return (jnp.where(ok, mid, lo), jnp.where(ok, hi, mid),
                        jnp.where(ok, cm, c))

            lo, _, _ = lax.while_loop(
                cond, body, (jnp.uint32(0), jnp.uint32(0xFFFFFFFF), c0))
            return jnp.full((16,), lo, jnp.uint32)

        def compress_into(src_v, src_i, dst_v, dst_i, dcap, t_u, cnt_v):
            nj = (jnp.max(cnt_v) + 15) >> 4

            def cpbody(j, newcnt):
                x = src_v[pl.ds(j * 16, 16)]
                col = src_i[pl.ds(j * 16, 16)]
                valid = (j * 16 + lanes) < cnt_v
                m = jnp.logical_and(_mono_u32(x) > t_u, valid)
                ones = jnp.where(m, 1, 0).astype(jnp.int32)
                incl = plsc.cumsum(ones)
                pos = jnp.minimum(newcnt + incl - 1, dcap - 1)
                plsc.store_scatter(dst_v, [pos], x, mask=m)
                plsc.store_scatter(dst_i, [pos], col, mask=m)
                return jnp.minimum(
                    newcnt + plsc.all_reduce_population_count(m),
                    _splat_i32(dcap))
            return lax.fori_loop(0, nj, cpbody, _splat_i32(0))

        def make_compact(r):
            def compact(cnt_v, tvec):
                t_u = find_threshold(cvs[r], cnt_v, 128)
                newcnt = compress_into(cvs[r], cis[r], cvs[r], cis[r],
                                       CAP, t_u, cnt_v)
                bv = t_u ^ jnp.where(t_u >= jnp.uint32(0x80000000),
                                     jnp.uint32(0x80000000),
                                     jnp.uint32(0xFFFFFFFF))
                return newcnt, plsc.bitcast(bv, jnp.float32)
            return compact

        compact_fns = (make_compact(0), make_compact(1))

        def scan_chunk(win, r, sub, cw, col_v, nv, st):
            cnt_v, tvec = st
            anym = win[sub, pl.ds(cw, 16)] > tvec
            for i in range(1, nv):
                anym = jnp.logical_or(
                    anym, win[sub, pl.ds(cw + i * 16, 16)] > tvec)

            def slow(cnt_v):
                for i in range(nv):
                    x = win[sub, pl.ds(cw + i * 16, 16)]
                    m = x > tvec
                    ones = jnp.where(m, 1, 0).astype(jnp.int32)
                    incl = plsc.cumsum(ones)
                    pos = jnp.minimum(cnt_v + incl - 1, _splat_i32(CAP + 15))
                    plsc.store_scatter(cvs[r], [pos], x, mask=m)
                    plsc.store_scatter(cis[r], [pos], col_v + i * 16, mask=m)
                    cnt_v = jnp.minimum(
                        cnt_v + plsc.all_reduce_population_count(m),
                        _splat_i32(CAP))
                return cnt_v

            cnt_v = lax.cond(jnp.any(anym), slow, lambda c: c, cnt_v)
            cnt_v, tvec = lax.cond(
                cnt_v[0] >= CAP - 16 * nv, compact_fns[r],
                lambda c, t: (c, t), cnt_v, tvec)
            return cnt_v, tvec

        def scan_tail(buf, r, col_v, st):
            cnt_v, tvec = st
            anym = buf[pl.ds(0, 16)] > tvec
            for i in range(1, TAILC // 16):
                anym = jnp.logical_or(anym, buf[pl.ds(i * 16, 16)] > tvec)

            def slow(cnt_v):
                for i in range(TAILC // 16):
                    x = buf[pl.ds(i * 16, 16)]
                    m = x > tvec
                    ones = jnp.where(m, 1, 0).astype(jnp.int32)
                    incl = plsc.cumsum(ones)
                    pos = jnp.minimum(cnt_v + incl - 1, _splat_i32(CAP + 15))
                    plsc.store_scatter(cvs[r], [pos], x, mask=m)
                    plsc.store_scatter(cis[r], [pos], col_v + i * 16, mask=m)
                    cnt_v = jnp.minimum(
                        cnt_v + plsc.all_reduce_population_count(m),
                        _splat_i32(CAP))
                return cnt_v

            cnt_v = lax.cond(jnp.any(anym), slow, lambda c: c, cnt_v)
            cnt_v, tvec = lax.cond(
                cnt_v[0] >= CAP - TAILC, compact_fns[r],
                lambda c, t: (c, t), cnt_v, tvec)
            return cnt_v, tvec

        def scan_window(win, c0, st0, st1):
            def tbody(t, carry):
                st0, st1 = carry
                col_v = c0 + t * 128 + lanes
                st0 = scan_chunk(win, 0, sub0, t * 128, col_v, 8, st0)
                st1 = scan_chunk(win, 1, sub0 + 1, t * 128, col_v, 8, st1)
                return st0, st1
            return lax.fori_loop(0, CT, tbody, (st0, st1))

        def src_at(w):
            c0 = pl.multiple_of(w * WC, 128)
            return logits_hbm.at[pl.ds(g8, 8), pl.ds(c0, WC)]

        pltpu.make_async_copy(src_at(0), win0, sem0).start()

        def wbody(i, carry):
            st0, st1 = carry
            w0 = 2 * i
            pltpu.make_async_copy(src_at(w0 + 1), win1, sem1).start()
            pltpu.make_async_copy(src_at(w0), win0, sem0).wait()
            st0, st1 = scan_window(win0, w0 * WC, st0, st1)

            @pl.when(i < NWIN_PAIRS - 1)
            def _():
                pltpu.make_async_copy(src_at(w0 + 2), win0, sem0).start()

            pltpu.make_async_copy(src_at(w0 + 1), win1, sem1).wait()
            st0, st1 = scan_window(win1, (w0 + 1) * WC, st0, st1)
            return st0, st1

        init = ((_splat_i32(0), _splat_f32(NEG_INF)),
                (_splat_i32(0), _splat_f32(NEG_INF)))
        (st0, st1) = lax.fori_loop(0, NWIN_PAIRS, wbody, init)

        tcol = NFULL * WC + lanes
        pltpu.sync_copy(tail_hbm.at[pl.ds(row0 * TAILC, TAILC)], wtail)
        st0 = scan_tail(wtail, 0, tcol, st0)
        pltpu.sync_copy(tail_hbm.at[pl.ds((row0 + 1) * TAILC, TAILC)], wtail)
        st1 = scan_tail(wtail, 1, tcol, st1)

        for r in range(2):
            row = row0 + r
            cnt_v = (st0, st1)[r][0]
            t_u = find_threshold(cvs[r], cnt_v, FCAP - 8)
            for i in range(FCAP // 16):
                fv[pl.ds(i * 16, 16)] = _splat_f32(NEG_INF)
                fi[pl.ds(i * 16, 16)] = _splat_i32(0)
            compress_into(cvs[r], cis[r], fv, fi, FCAP, t_u, cnt_v)
            for i in range(KPAD // 16):
                ov[pl.ds(i * 16, 16)] = _splat_f32(NEG_INF)
                oi[pl.ds(i * 16, 16)] = _splat_i32(0)

            def ebody(j, _):
                mvec = fv[pl.ds(0, 16)]
                for i in range(1, FCAP // 16):
                    mvec = jnp.maximum(mvec, fv[pl.ds(i * 16, 16)])
                msp = _splat_f32(1.0) * jnp.max(mvec)
                pos_v = _splat_i32(FCAP)
                for i in range(FCAP // 16):
                    eq = fv[pl.ds(i * 16, 16)] == msp
                    pos_v = jnp.minimum(
                        pos_v, jnp.where(eq, i * 16 + lanes, FCAP))
                pos = _splat_i32(1) * jnp.min(pos_v)
                pos = jnp.minimum(pos, _splat_i32(FCAP - 1))
                idv = plsc.load_gather(fi, [pos])
                jsp = _splat_i32(1) * j
                lane0 = lanes == 0
                plsc.store_scatter(ov, [jsp], msp, mask=lane0)
                plsc.store_scatter(oi, [jsp], idv, mask=lane0)
                plsc.store_scatter(fv, [pos], _splat_f32(NEG_INF),
                                   mask=lane0)
                return 0
            lax.fori_loop(0, K, ebody, 0)

            pltpu.sync_copy(gumbel_hbm.at[pl.ds(row * KPAD, KPAD)], gb)
            zbest = _splat_f32(NEG_INF)
            zs = []
            for i in range(KPAD // 16):
                z = ov[pl.ds(i * 16, 16)] + gb[pl.ds(i * 16, 16)]
                zs.append(z)
                zbest = jnp.maximum(zbest, z)
            msp = _splat_f32(1.0) * jnp.max(zbest)
            pos_v = _splat_i32(KPAD)
            for i in range(KPAD // 16):
                eq = zs[i] == msp
                pos_v = jnp.minimum(pos_v,
                                    jnp.where(eq, i * 16 + lanes, KPAD))
            pos = _splat_i32(1) * jnp.min(pos_v)
            pos = jnp.minimum(pos, _splat_i32(KPAD - 1))
            actb[...] = plsc.load_gather(oi, [pos])

            pltpu.sync_copy(ov, vals_hbm.at[pl.ds(row * KPAD, KPAD)])
            pltpu.sync_copy(actb, act_hbm.at[pl.ds(row * ACTW, ACTW)])

    return k(logits2d, tail_flat, gumbel_flat)


def kernel(logits):
    gkey = jax.random.key(42)
    u = jax.random.uniform(gkey, (B, K), minval=1e-20, maxval=1.0)
    gumbel = -jnp.log(-jnp.log(u))
    gpad = jnp.full((B, KPAD), NEG_INF, dtype=jnp.float32)
    gpad = gpad.at[:, :K].set(gumbel)

    tail_flat = logits[:, NFULL * WC:].reshape(-1)
    vals_flat, act_flat = _sc_topk_sample(logits, tail_flat,
                                          gpad.reshape(-1))
    vals = vals_flat.reshape(B, KPAD)[:, :K]
    act = act_flat.reshape(B, ACTW)[:, 0]
    return act, vals
